# baseline (speedup 1.0000x reference)
import functools

import jax
import jax.numpy as jnp
from jax import lax
from jax.experimental import pallas as pl
from jax.experimental.pallas import tpu as pltpu
from jax.experimental.pallas import tpu_sc as plsc

N = 10000
E = 320000
D = 128
NPAD = 10240
NC = 2
NS = 16
NW = NC * NS
EPT = E // NW
CHUNK = 80
ITERS = EPT // CHUNK
SCH = 80
NBUF = 3
SITERS = EPT // SCH
SGROUPS = SITERS // NBUF
STAIL = SITERS - SGROUPS * NBUF
NACC = N
APS = N // NS
WCH = 125
RPS = NPAD // NS
BLK = 2048


def _sc_mesh():
    return plsc.VectorSubcoreMesh(
        core_axis_name="c", subcore_axis_name="s", num_cores=NC, num_subcores=NS
    )


def _deg_partials(dst_r):
    @functools.partial(
        pl.kernel,
        out_type=jax.ShapeDtypeStruct((NC, NPAD), jnp.float32),
        mesh=_sc_mesh(),
        scratch_types=[
            pltpu.VMEM((ITERS, CHUNK), jnp.int32),
            pltpu.VMEM((CHUNK,), jnp.float32),
            pltpu.VMEM((RPS,), jnp.float32),
            pltpu.VMEM_SHARED((NPAD,), jnp.float32),
            pltpu.SemaphoreType.DMA,
        ],
    )
    def deg_kernel(dst_hbm, out_hbm, dst_v, ones_v, zb, hist, sem):
        c = lax.axis_index("c")
        s = lax.axis_index("s")
        wid = c * NS + s
        pltpu.sync_copy(dst_hbm.at[wid], dst_v)

        ones = jnp.ones((16,), jnp.float32)
        zv = jnp.zeros((16,), jnp.float32)
        for k in range(CHUNK // 16):
            ones_v[pl.ds(k * 16, 16)] = ones

        def zero_row(r, _):
            zb[pl.ds(r * 16, 16)] = zv
            return 0

        lax.fori_loop(0, RPS // 16, zero_row, 0)
        pltpu.sync_copy(zb, hist.at[pl.ds(s * RPS, RPS)])
        plsc.subcore_barrier()

        def issue(j, _):
            pltpu.async_copy(ones_v, hist.at[dst_v.at[j]], sem, add=True)
            return 0

        lax.fori_loop(0, ITERS, issue, 0)

        def drain(j, _):
            pltpu.make_async_copy(ones_v, hist.at[dst_v.at[0]], sem).wait()
            return 0

        lax.fori_loop(0, ITERS, drain, 0)
        plsc.subcore_barrier()
        pltpu.sync_copy(hist.at[pl.ds(s * RPS, RPS)], out_hbm.at[c, pl.ds(s * RPS, RPS)])

    return deg_kernel(dst_r)


def _scatter_partials(hw, src_r, dst_r):
    @functools.partial(
        pl.kernel,
        out_type=jax.ShapeDtypeStruct((NC, NPAD, D), jnp.float32),
        mesh=_sc_mesh(),
        scratch_types=[
            pltpu.VMEM((SITERS, SCH), jnp.int32),
            pltpu.VMEM((SITERS, SCH), jnp.int32),
            pltpu.VMEM((NBUF * SCH, D), jnp.float32),
            pltpu.VMEM_SHARED((NACC, D), jnp.float32),
            pltpu.SemaphoreType.DMA((NBUF,)),
            pltpu.SemaphoreType.DMA((NBUF,)),
        ],
        compiler_params=pltpu.CompilerParams(use_tc_tiling_on_sc=False),
    )
    def scat_kernel(hw_hbm, src_hbm, dst_hbm, out_hbm,
                    src_v, dst_v, rows_v, acc, gsem, ssem):
        c = lax.axis_index("c")
        s = lax.axis_index("s")
        wid = c * NS + s
        pltpu.sync_copy(src_hbm.at[wid], src_v)
        pltpu.sync_copy(dst_hbm.at[wid], dst_v)

        zv = jnp.zeros((16,), jnp.float32)

        def zero_row(r, _):
            for jj in range(D // 16):
                rows_v[r, pl.ds(jj * 16, 16)] = zv
            return 0

        lax.fori_loop(0, WCH, zero_row, 0)
        for k in range(APS // WCH):
            pltpu.sync_copy(rows_v.at[pl.ds(0, WCH)],
                            acc.at[pl.ds(s * APS + k * WCH, WCH)])
        plsc.subcore_barrier()

        def slot(b):
            return rows_v.at[pl.ds(b * SCH, SCH)]

        def start_gather(j, b):
            pltpu.async_copy(hw_hbm.at[src_v.at[j]], slot(b), gsem.at[b])

        def wait_gather(b):
            pltpu.make_async_copy(
                hw_hbm.at[src_v.at[0]], slot(b), gsem.at[b]
            ).wait()

        def start_scatter(j, b):
            pltpu.async_copy(slot(b), acc.at[dst_v.at[j]], ssem.at[b], add=True)

        def wait_scatter(b):
            pltpu.make_async_copy(
                slot(b), acc.at[dst_v.at[0]], ssem.at[b]
            ).wait()

        for b in range(NBUF):
            start_gather(b, b)

        def group(g, _):
            for b in range(NBUF):
                wait_gather(b)
                start_scatter(g * NBUF + b, b)
            for b in range(NBUF):
                wait_scatter(b)
                jn = (g + 1) * NBUF + b

                @pl.when(jn < SITERS)
                def _():
                    start_gather(jn, b)

            return 0

        lax.fori_loop(0, SGROUPS - 1, group, 0)
        for b in range(NBUF):
            wait_gather(b)
            start_scatter((SGROUPS - 1) * NBUF + b, b)
        for b in range(NBUF):
            wait_scatter(b)
            if b < STAIL:
                start_gather(SGROUPS * NBUF + b, b)
        for b in range(STAIL):
            wait_gather(b)
            start_scatter(SGROUPS * NBUF + b, b)
        for b in range(STAIL):
            wait_scatter(b)
        plsc.subcore_barrier()

        for k in range(APS // WCH):
            r0 = s * APS + k * WCH
            pltpu.sync_copy(acc.at[pl.ds(r0, WCH)], out_hbm.at[c, pl.ds(r0, WCH)])

    return scat_kernel(hw, src_r, dst_r)


def _tc_first(x, deg2, W):
    def body(x_ref, d_ref, w_ref, o_ref):
        deg = d_ref[0] + d_ref[1]
        nrm = lax.rsqrt(deg + 1.0)
        o_ref[...] = jnp.dot(
            x_ref[...] * nrm, w_ref[...], preferred_element_type=jnp.float32
        )

    return pl.pallas_call(
        body,
        out_shape=jax.ShapeDtypeStruct((NPAD, D), jnp.float32),
        grid=(NPAD // BLK,),
        in_specs=[
            pl.BlockSpec((BLK, D), lambda i: (i, 0)),
            pl.BlockSpec((2, BLK, 1), lambda i: (0, i, 0)),
            pl.BlockSpec((D, D), lambda i: (0, 0)),
        ],
        out_specs=pl.BlockSpec((BLK, D), lambda i: (i, 0)),
    )(x, deg2, W)


def _tc_mid(p, hw, deg2, Wn, b, l):
    lf = float(l)

    def body(p_ref, hw_ref, d_ref, w_ref, b_ref, o_ref):
        deg = d_ref[0] + d_ref[1]
        nrm_l = lax.rsqrt(deg + lf)
        nrm_n = lax.rsqrt(deg + lf + 1.0)
        agg = p_ref[0] + p_ref[1] + lf * hw_ref[...]
        h = jnp.maximum(agg * nrm_l + b_ref[...], 0.0)
        o_ref[...] = jnp.dot(
            h * nrm_n, w_ref[...], preferred_element_type=jnp.float32
        )

    return pl.pallas_call(
        body,
        out_shape=jax.ShapeDtypeStruct((NPAD, D), jnp.float32),
        grid=(NPAD // BLK,),
        in_specs=[
            pl.BlockSpec((2, BLK, D), lambda i: (0, i, 0)),
            pl.BlockSpec((BLK, D), lambda i: (i, 0)),
            pl.BlockSpec((2, BLK, 1), lambda i: (0, i, 0)),
            pl.BlockSpec((D, D), lambda i: (0, 0)),
            pl.BlockSpec((1, D), lambda i: (0, 0)),
        ],
        out_specs=pl.BlockSpec((BLK, D), lambda i: (i, 0)),
    )(p, hw, deg2, Wn, b)


def _tc_final(p, hw, deg2, b):
    def body(p_ref, hw_ref, d_ref, b_ref, o_ref):
        deg = d_ref[0] + d_ref[1]
        nrm = lax.rsqrt(deg + 3.0)
        agg = p_ref[0] + p_ref[1] + 3.0 * hw_ref[...]
        o_ref[...] = agg * nrm + b_ref[...]

    return pl.pallas_call(
        body,
        out_shape=jax.ShapeDtypeStruct((NPAD, D), jnp.float32),
        grid=(NPAD // BLK,),
        in_specs=[
            pl.BlockSpec((2, BLK, D), lambda i: (0, i, 0)),
            pl.BlockSpec((BLK, D), lambda i: (i, 0)),
            pl.BlockSpec((2, BLK, 1), lambda i: (0, i, 0)),
            pl.BlockSpec((1, D), lambda i: (0, 0)),
        ],
        out_specs=pl.BlockSpec((BLK, D), lambda i: (i, 0)),
    )(p, hw, deg2, b)


def kernel(features, edge_index, W1, b1, W2, b2, W3, b3):
    dst_r = edge_index[1].reshape(NW, ITERS, CHUNK)
    src_s = edge_index[0].reshape(NW, SITERS, SCH)
    dst_s = edge_index[1].reshape(NW, SITERS, SCH)
    x = jnp.pad(features, ((0, NPAD - N), (0, 0)))
    b1r, b2r, b3r = b1.reshape(1, D), b2.reshape(1, D), b3.reshape(1, D)

    deg2 = _deg_partials(dst_r).reshape(NC, NPAD, 1)
    hw1 = _tc_first(x, deg2, W1)
    p1 = _scatter_partials(hw1, src_s, dst_s)
    hw2 = _tc_mid(p1, hw1, deg2, W2, b1r, 1)
    p2 = _scatter_partials(hw2, src_s, dst_s)
    hw3 = _tc_mid(p2, hw2, deg2, W3, b2r, 2)
    p3 = _scatter_partials(hw3, src_s, dst_s)
    out = _tc_final(p3, hw3, deg2, b3r)
    return out[:N]


# speedup vs baseline: 1.1283x; 1.1283x over previous
import functools

import jax
import jax.numpy as jnp
from jax import lax
from jax.experimental import pallas as pl
from jax.experimental.pallas import tpu as pltpu
from jax.experimental.pallas import tpu_sc as plsc

N = 10000
E = 320000
D = 128
NPAD = 10240
NC = 2
NS = 16
NW = NC * NS
EPT = E // NW
CHUNK = 80
ITERS = EPT // CHUNK
RPS = NPAD // NS
SCH = 40
NBUF = 6
SITERS = EPT // SCH
SGROUPS = SITERS // NBUF
STAIL = SITERS - SGROUPS * NBUF
APS = N // NS
WCH = 125
BLK = 2000


def _sc_mesh():
    return plsc.VectorSubcoreMesh(
        core_axis_name="c", subcore_axis_name="s", num_cores=NC, num_subcores=NS
    )


def _deg_partials(dst_r):
    @functools.partial(
        pl.kernel,
        out_type=jax.ShapeDtypeStruct((NC, NPAD), jnp.float32),
        mesh=_sc_mesh(),
        scratch_types=[
            pltpu.VMEM((ITERS, CHUNK), jnp.int32),
            pltpu.VMEM((CHUNK,), jnp.float32),
            pltpu.VMEM((RPS,), jnp.float32),
            pltpu.VMEM_SHARED((NPAD,), jnp.float32),
            pltpu.SemaphoreType.DMA,
        ],
    )
    def deg_kernel(dst_hbm, out_hbm, dst_v, ones_v, zb, hist, sem):
        c = lax.axis_index("c")
        s = lax.axis_index("s")
        wid = c * NS + s
        pltpu.sync_copy(dst_hbm.at[wid], dst_v)

        ones = jnp.ones((16,), jnp.float32)
        zv = jnp.zeros((16,), jnp.float32)
        for k in range(CHUNK // 16):
            ones_v[pl.ds(k * 16, 16)] = ones

        def zero_row(r, _):
            zb[pl.ds(r * 16, 16)] = zv
            return 0

        lax.fori_loop(0, RPS // 16, zero_row, 0)
        pltpu.sync_copy(zb, hist.at[pl.ds(s * RPS, RPS)])
        plsc.subcore_barrier()

        def issue(j, _):
            pltpu.async_copy(ones_v, hist.at[dst_v.at[j]], sem, add=True)
            return 0

        lax.fori_loop(0, ITERS, issue, 0)

        def drain(j, _):
            pltpu.make_async_copy(ones_v, hist.at[dst_v.at[0]], sem).wait()
            return 0

        lax.fori_loop(0, ITERS, drain, 0)
        plsc.subcore_barrier()
        pltpu.sync_copy(hist.at[pl.ds(s * RPS, RPS)], out_hbm.at[c, pl.ds(s * RPS, RPS)])

    return deg_kernel(dst_r)


def _scatter_partials(hw, src_r, dst_r):
    @functools.partial(
        pl.kernel,
        out_type=jax.ShapeDtypeStruct((NC, N, D), jnp.float32),
        mesh=_sc_mesh(),
        scratch_types=[
            pltpu.VMEM((SITERS, SCH), jnp.int32),
            pltpu.VMEM((SITERS, SCH), jnp.int32),
            pltpu.VMEM((NBUF * SCH, D), jnp.float32),
            pltpu.VMEM_SHARED((N, D), jnp.float32),
            pltpu.SemaphoreType.DMA((NBUF,)),
            pltpu.SemaphoreType.DMA((NBUF,)),
        ],
        compiler_params=pltpu.CompilerParams(use_tc_tiling_on_sc=False),
    )
    def scat_kernel(hw_hbm, src_hbm, dst_hbm, out_hbm,
                    src_v, dst_v, rows_v, acc, gsem, ssem):
        c = lax.axis_index("c")
        s = lax.axis_index("s")
        wid = c * NS + s
        pltpu.sync_copy(src_hbm.at[wid], src_v)
        pltpu.sync_copy(dst_hbm.at[wid], dst_v)

        zv = jnp.zeros((16,), jnp.float32)

        def zero_row(r, _):
            for jj in range(D // 16):
                rows_v[r, pl.ds(jj * 16, 16)] = zv
            return 0

        lax.fori_loop(0, WCH, zero_row, 0)
        for k in range(APS // WCH):
            pltpu.sync_copy(rows_v.at[pl.ds(0, WCH)],
                            acc.at[pl.ds(s * APS + k * WCH, WCH)])
        plsc.subcore_barrier()

        def slot(b):
            return rows_v.at[pl.ds(b * SCH, SCH)]

        def start_gather(j, b):
            pltpu.async_copy(hw_hbm.at[src_v.at[j]], slot(b), gsem.at[b])

        def wait_gather(b):
            pltpu.make_async_copy(
                hw_hbm.at[src_v.at[0]], slot(b), gsem.at[b]
            ).wait()

        def start_scatter(j, b):
            pltpu.async_copy(slot(b), acc.at[dst_v.at[j]], ssem.at[b], add=True)

        def wait_scatter(b):
            pltpu.make_async_copy(
                slot(b), acc.at[dst_v.at[0]], ssem.at[b]
            ).wait()

        for b in range(NBUF):
            start_gather(b, b)

        def group(g, _):
            for b in range(NBUF):
                wait_gather(b)
                start_scatter(g * NBUF + b, b)
            for b in range(NBUF):
                wait_scatter(b)
                start_gather((g + 1) * NBUF + b, b)
            return 0

        lax.fori_loop(0, SGROUPS - 1, group, 0)
        for b in range(NBUF):
            wait_gather(b)
            start_scatter((SGROUPS - 1) * NBUF + b, b)
        for b in range(NBUF):
            wait_scatter(b)
            if b < STAIL:
                start_gather(SGROUPS * NBUF + b, b)
        for b in range(STAIL):
            wait_gather(b)
            start_scatter(SGROUPS * NBUF + b, b)
        for b in range(STAIL):
            wait_scatter(b)
        plsc.subcore_barrier()

        for k in range(APS // WCH):
            r0 = s * APS + k * WCH
            pltpu.sync_copy(acc.at[pl.ds(r0, WCH)], out_hbm.at[c, pl.ds(r0, WCH)])

    return scat_kernel(hw, src_r, dst_r)


def _tc_first(x, deg2, W):
    def body(x_ref, d_ref, w_ref, o_ref):
        deg = d_ref[0] + d_ref[1]
        nrm = lax.rsqrt(deg + 1.0)
        o_ref[...] = jnp.dot(
            x_ref[...] * nrm, w_ref[...], preferred_element_type=jnp.float32
        )

    return pl.pallas_call(
        body,
        out_shape=jax.ShapeDtypeStruct((N, D), jnp.float32),
        grid=(N // BLK,),
        in_specs=[
            pl.BlockSpec((BLK, D), lambda i: (i, 0)),
            pl.BlockSpec((2, BLK, 1), lambda i: (0, i, 0)),
            pl.BlockSpec((D, D), lambda i: (0, 0)),
        ],
        out_specs=pl.BlockSpec((BLK, D), lambda i: (i, 0)),
    )(x, deg2, W)


def _tc_mid(p, hw, deg2, Wn, b, l):
    lf = float(l)

    def body(p_ref, hw_ref, d_ref, w_ref, b_ref, o_ref):
        deg = d_ref[0] + d_ref[1]
        nrm_l = lax.rsqrt(deg + lf)
        nrm_n = lax.rsqrt(deg + lf + 1.0)
        agg = p_ref[0] + p_ref[1] + lf * hw_ref[...]
        h = jnp.maximum(agg * nrm_l + b_ref[...], 0.0)
        o_ref[...] = jnp.dot(
            h * nrm_n, w_ref[...], preferred_element_type=jnp.float32
        )

    return pl.pallas_call(
        body,
        out_shape=jax.ShapeDtypeStruct((N, D), jnp.float32),
        grid=(N // BLK,),
        in_specs=[
            pl.BlockSpec((2, BLK, D), lambda i: (0, i, 0)),
            pl.BlockSpec((BLK, D), lambda i: (i, 0)),
            pl.BlockSpec((2, BLK, 1), lambda i: (0, i, 0)),
            pl.BlockSpec((D, D), lambda i: (0, 0)),
            pl.BlockSpec((1, D), lambda i: (0, 0)),
        ],
        out_specs=pl.BlockSpec((BLK, D), lambda i: (i, 0)),
    )(p, hw, deg2, Wn, b)


def _tc_final(p, hw, deg2, b):
    def body(p_ref, hw_ref, d_ref, b_ref, o_ref):
        deg = d_ref[0] + d_ref[1]
        nrm = lax.rsqrt(deg + 3.0)
        agg = p_ref[0] + p_ref[1] + 3.0 * hw_ref[...]
        o_ref[...] = agg * nrm + b_ref[...]

    return pl.pallas_call(
        body,
        out_shape=jax.ShapeDtypeStruct((N, D), jnp.float32),
        grid=(N // BLK,),
        in_specs=[
            pl.BlockSpec((2, BLK, D), lambda i: (0, i, 0)),
            pl.BlockSpec((BLK, D), lambda i: (i, 0)),
            pl.BlockSpec((2, BLK, 1), lambda i: (0, i, 0)),
            pl.BlockSpec((1, D), lambda i: (0, 0)),
        ],
        out_specs=pl.BlockSpec((BLK, D), lambda i: (i, 0)),
    )(p, hw, deg2, b)


def kernel(features, edge_index, W1, b1, W2, b2, W3, b3):
    dst_deg = edge_index[1].reshape(NW, ITERS, CHUNK)
    src_s = edge_index[0].reshape(NW, SITERS, SCH)
    dst_s = edge_index[1].reshape(NW, SITERS, SCH)
    b1r, b2r, b3r = b1.reshape(1, D), b2.reshape(1, D), b3.reshape(1, D)

    deg2 = _deg_partials(dst_deg).reshape(NC, NPAD, 1)
    hw1 = _tc_first(features, deg2, W1)
    p1 = _scatter_partials(hw1, src_s, dst_s)
    hw2 = _tc_mid(p1, hw1, deg2, W2, b1r, 1)
    p2 = _scatter_partials(hw2, src_s, dst_s)
    hw3 = _tc_mid(p2, hw2, deg2, W3, b2r, 2)
    p3 = _scatter_partials(hw3, src_s, dst_s)
    return _tc_final(p3, hw3, deg2, b3r)
